# baseline (speedup 1.0000x reference)
import dataclasses
import functools

import jax
import jax.numpy as jnp
from jax import lax
from jax.experimental import pallas as pl
from jax.experimental.pallas import tpu as pltpu
from jax.experimental.pallas import tpu_sc as plsc

_VOCAB = 1000
_DIM = 64
_BATCH = 1024
_SEQ = 50

_NC = 2
_NS = 16
_DSLAB = 8
_NSLAB = _DIM // _DSLAB
_SGRP = 13
_LANES = 16
_UNROLL = 2


def kernel(token_ids, W):
    tok_t = token_ids.T.reshape(_SEQ, 1, _BATCH)
    w_t = W.T
    mesh = plsc.VectorSubcoreMesh(core_axis_name="c", subcore_axis_name="s")
    cp = pltpu.CompilerParams()
    if "needs_layout_passes" in pltpu.CompilerParams.__dataclass_fields__:
        cp = dataclasses.replace(cp, needs_layout_passes=False)

    @functools.partial(
        pl.kernel,
        mesh=mesh,
        compiler_params=cp,
        out_type=jax.ShapeDtypeStruct((_SEQ, _DIM, _BATCH), W.dtype),
        scratch_types=[
            pltpu.VMEM((_DSLAB, _VOCAB), jnp.float32),
            pltpu.VMEM((_SGRP, 1, _BATCH), jnp.int32),
            pltpu.VMEM((1, _DSLAB, _BATCH), jnp.float32),
            pltpu.VMEM((1, _DSLAB, _BATCH), jnp.float32),
            pltpu.SemaphoreType.DMA,
            pltpu.SemaphoreType.DMA,
        ],
    )
    def emb_kernel(w_hbm, tok_hbm, out_hbm, wbuf, tokbuf, ob_a, ob_b, sw_a, sw_b):
        wid = lax.axis_index("s") * _NC + lax.axis_index("c")
        dslab = wid % _NSLAB
        d0 = dslab * _DSLAB
        grp = wid // _NSLAB
        s0 = jnp.minimum(grp * _SGRP, _SEQ - _SGRP)

        pltpu.sync_copy(w_hbm.at[pl.ds(d0, _DSLAB)], wbuf)
        pltpu.sync_copy(tok_hbm.at[pl.ds(s0, _SGRP)], tokbuf)

        obufs = (ob_a, ob_b)
        sws = (sw_a, sw_b)
        dvecs = [jnp.full((_LANES,), d, jnp.int32) for d in range(_DSLAB)]

        def compute(si, ob):
            @pl.loop(0, _BATCH // (_LANES * _UNROLL))
            def _(bgo):
                tvs = [
                    tokbuf[si, 0, pl.ds((bgo * _UNROLL + u) * _LANES, _LANES)]
                    for u in range(_UNROLL)
                ]
                vals = [
                    [plsc.load_gather(wbuf, [dvecs[d], tvs[u]]) for d in range(_DSLAB)]
                    for u in range(_UNROLL)
                ]
                for u in range(_UNROLL):
                    b0 = (bgo * _UNROLL + u) * _LANES
                    for d in range(_DSLAB):
                        ob[0, d, pl.ds(b0, _LANES)] = vals[u][d]

        def wb_start(si, b):
            pltpu.make_async_copy(
                obufs[b],
                out_hbm.at[pl.ds(s0 + si, 1), pl.ds(d0, _DSLAB)],
                sws[b],
            ).start()

        def wb_wait(si, b):
            pltpu.make_async_copy(
                obufs[b],
                out_hbm.at[pl.ds(s0 + si, 1), pl.ds(d0, _DSLAB)],
                sws[b],
            ).wait()

        compute(0, obufs[0])
        wb_start(0, 0)
        compute(1, obufs[1])
        wb_start(1, 1)

        @pl.loop(2, _SGRP - 1, step=2)
        def _(si):
            wb_wait(si - 2, 0)
            compute(si, obufs[0])
            wb_start(si, 0)
            wb_wait(si - 1, 1)
            compute(si + 1, obufs[1])
            wb_start(si + 1, 1)

        wb_wait(_SGRP - 3, 0)
        compute(_SGRP - 1, obufs[0])
        wb_start(_SGRP - 1, 0)
        wb_wait(_SGRP - 2, 1)
        wb_wait(_SGRP - 1, 0)

    out = emb_kernel(w_t, tok_t)
    return out.transpose(2, 0, 1)


# speedup vs baseline: 1.0513x; 1.0513x over previous
import dataclasses
import functools

import jax
import jax.numpy as jnp
from jax import lax
from jax.experimental import pallas as pl
from jax.experimental.pallas import tpu as pltpu
from jax.experimental.pallas import tpu_sc as plsc

_VOCAB = 1000
_DIM = 64
_BATCH = 1024
_SEQ = 50

_NC = 2
_NS = 16
_DSLAB = 8
_NSLAB = _DIM // _DSLAB
_SGRP = 13
_LANES = 16
_UNROLL = 8


def kernel(token_ids, W):
    tok_t = token_ids.T.reshape(_SEQ, 1, _BATCH)
    w_t = W.T
    mesh = plsc.VectorSubcoreMesh(core_axis_name="c", subcore_axis_name="s")
    cp = pltpu.CompilerParams()
    if "needs_layout_passes" in pltpu.CompilerParams.__dataclass_fields__:
        cp = dataclasses.replace(cp, needs_layout_passes=False)

    @functools.partial(
        pl.kernel,
        mesh=mesh,
        compiler_params=cp,
        out_type=jax.ShapeDtypeStruct((_SEQ, _DIM, _BATCH), W.dtype),
        scratch_types=[
            pltpu.VMEM((_DSLAB, _VOCAB), jnp.float32),
            pltpu.VMEM((_SGRP, 1, _BATCH), jnp.int32),
            pltpu.VMEM((1, _DSLAB, _BATCH), jnp.float32),
            pltpu.VMEM((1, _DSLAB, _BATCH), jnp.float32),
            pltpu.SemaphoreType.DMA,
            pltpu.SemaphoreType.DMA,
        ],
    )
    def emb_kernel(w_hbm, tok_hbm, out_hbm, wbuf, tokbuf, ob_a, ob_b, sw_a, sw_b):
        wid = lax.axis_index("s") * _NC + lax.axis_index("c")
        dslab = wid % _NSLAB
        d0 = dslab * _DSLAB
        grp = wid // _NSLAB
        s0 = jnp.minimum(grp * _SGRP, _SEQ - _SGRP)

        pltpu.sync_copy(w_hbm.at[pl.ds(d0, _DSLAB)], wbuf)
        pltpu.sync_copy(tok_hbm.at[pl.ds(s0, _SGRP)], tokbuf)

        obufs = (ob_a, ob_b)
        sws = (sw_a, sw_b)
        dvecs = [jnp.full((_LANES,), d, jnp.int32) for d in range(_DSLAB)]

        def compute(si, ob):
            @pl.loop(0, _BATCH // (_LANES * _UNROLL))
            def _(bgo):
                tvs = [
                    tokbuf[si, 0, pl.ds((bgo * _UNROLL + u) * _LANES, _LANES)]
                    for u in range(_UNROLL)
                ]
                vals = [
                    [plsc.load_gather(wbuf, [dvecs[d], tvs[u]]) for d in range(_DSLAB)]
                    for u in range(_UNROLL)
                ]
                for u in range(_UNROLL):
                    b0 = (bgo * _UNROLL + u) * _LANES
                    for d in range(_DSLAB):
                        ob[0, d, pl.ds(b0, _LANES)] = vals[u][d]

        def wb_start(si, b):
            pltpu.make_async_copy(
                obufs[b],
                out_hbm.at[pl.ds(s0 + si, 1), pl.ds(d0, _DSLAB)],
                sws[b],
            ).start()

        def wb_wait(si, b):
            pltpu.make_async_copy(
                obufs[b],
                out_hbm.at[pl.ds(s0 + si, 1), pl.ds(d0, _DSLAB)],
                sws[b],
            ).wait()

        compute(0, obufs[0])
        wb_start(0, 0)
        compute(1, obufs[1])
        wb_start(1, 1)

        @pl.loop(2, _SGRP - 1, step=2)
        def _(si):
            wb_wait(si - 2, 0)
            compute(si, obufs[0])
            wb_start(si, 0)
            wb_wait(si - 1, 1)
            compute(si + 1, obufs[1])
            wb_start(si + 1, 1)

        wb_wait(_SGRP - 3, 0)
        compute(_SGRP - 1, obufs[0])
        wb_start(_SGRP - 1, 0)
        wb_wait(_SGRP - 2, 1)
        wb_wait(_SGRP - 1, 0)

    out = emb_kernel(w_t, tok_t)
    return out.transpose(2, 0, 1)
